# baseline (speedup 1.0000x reference)
import functools

import jax
import jax.numpy as jnp
from jax import lax
from jax.experimental import pallas as pl
from jax.experimental.pallas import tpu as pltpu
from jax.experimental.pallas import tpu_sc as plsc

_BATCH = 16384
_LATENT = 64
_CHUNK = 128


def kernel(users, movies, U, M, bu, bm, b0):
    info = plsc.get_sparse_core_info()
    nc, ns, nl = info.num_cores, info.num_subcores, info.num_lanes
    nw = nc * ns
    bpw = _BATCH // nw
    nchunk = bpw // _CHUNK
    nck = 2 * _LATENT // nl

    mesh = plsc.VectorSubcoreMesh(core_axis_name="c", subcore_axis_name="s")

    users2 = users.astype(jnp.int32).reshape(nw * nchunk, _CHUNK)
    movies2 = movies.astype(jnp.int32).reshape(nw * nchunk, _CHUNK)
    U2 = U.reshape(U.shape[0] // 2, 2 * _LATENT)
    M2 = M.reshape(M.shape[0] // 2, 2 * _LATENT)
    b0v = jnp.broadcast_to(b0.astype(jnp.float32), (nl,))

    @functools.partial(
        pl.kernel,
        mesh=mesh,
        out_type=jax.ShapeDtypeStruct((_BATCH,), jnp.float32),
        scratch_types=[
            pltpu.VMEM((nchunk, _CHUNK), jnp.int32),
            pltpu.VMEM((nchunk, _CHUNK), jnp.int32),
            pltpu.VMEM((nchunk, _CHUNK), jnp.int32),
            pltpu.VMEM((nchunk, _CHUNK), jnp.int32),
            pltpu.VMEM((nchunk, _CHUNK), jnp.int32),
            pltpu.VMEM((nchunk, _CHUNK), jnp.int32),
            pltpu.VMEM((_CHUNK, 2 * _LATENT), jnp.float32),
            pltpu.VMEM((_CHUNK, 2 * _LATENT), jnp.float32),
            pltpu.VMEM((bpw,), jnp.float32),
            pltpu.VMEM((bpw,), jnp.float32),
            pltpu.VMEM((nl,), jnp.float32),
            pltpu.VMEM((bpw,), jnp.float32),
            pltpu.SemaphoreType.DMA,
            pltpu.SemaphoreType.DMA,
        ],
    )
    def run(users_h, movies_h, U_h, M_h, bu_h, bm_h, b0_h, out_h,
            uidx, midx, urow, mrow, upar, mpar, ub, mb,
            ubias, mbias, b0s, outv, semr, semb):
        wid = lax.axis_index("s") * nc + lax.axis_index("c")
        base = wid * bpw
        rbase = wid * nchunk

        pltpu.sync_copy(users_h.at[pl.ds(rbase, nchunk)], uidx)
        pltpu.sync_copy(movies_h.at[pl.ds(rbase, nchunk)], midx)
        pltpu.sync_copy(b0_h, b0s)

        bcopies = []
        for i in range(nchunk):
            sl = pl.ds(i * _CHUNK, _CHUNK)
            bcopies.append(pltpu.async_copy(bu_h.at[uidx.at[i]], ubias.at[sl], semb))
            bcopies.append(pltpu.async_copy(bm_h.at[midx.at[i]], mbias.at[sl], semb))

        for i in range(nchunk):
            for k in range(_CHUNK // nl):
                sl = pl.ds(k * nl, nl)
                v = uidx[i, sl]
                urow[i, sl] = v >> 1
                upar[i, sl] = (v & 1) * _LATENT
                v = midx[i, sl]
                mrow[i, sl] = v >> 1
                mpar[i, sl] = (v & 1) * _LATENT

        def issue(c):
            return [pltpu.async_copy(U_h.at[urow.at[c]], ub, semr),
                    pltpu.async_copy(M_h.at[mrow.at[c]], mb, semr)]

        for c in bcopies:
            c.wait()
        b0vec = b0s[...]

        even = jnp.arange(0, 2 * nl, 2, jnp.int32) % nl
        odd = even + 1
        lane_lo = lax.broadcasted_iota(jnp.int32, (nl,), 0) < (nl // 2)

        def shuf(a, idx):
            return a.at[idx].get(mode="promise_in_bounds")

        def hadd(a, b):
            ha = shuf(a, even) + shuf(a, odd)
            hb = shuf(b, even) + shuf(b, odd)
            return jnp.where(lane_lo, ha, hb)

        rcopies = issue(0)
        for c in range(nchunk):
            for cp in rcopies:
                cp.wait()

            def group(g, carry):
                gbase = g * nl
                pu = upar[c, pl.ds(gbase, nl)]
                pm = mpar[c, pl.ds(gbase, nl)]
                vecs = []
                for jj in range(nl):
                    j = gbase + jj
                    hu = pu[jj]
                    hm = pm[jj]
                    acc = ub[j, pl.ds(hu, nl)] * mb[j, pl.ds(hm, nl)]
                    for k in range(1, _LATENT // nl):
                        acc = acc + (ub[j, pl.ds(hu + k * nl, nl)]
                                     * mb[j, pl.ds(hm + k * nl, nl)])
                    vecs.append(acc)
                while len(vecs) > 1:
                    vecs = [hadd(vecs[t], vecs[t + 1])
                            for t in range(0, len(vecs), 2)]
                sl = pl.ds(c * _CHUNK + gbase, nl)
                r = vecs[0] + ubias[sl] + mbias[sl] + b0vec
                outv[sl] = 5.0 / (1.0 + jnp.exp(-r))
                return carry

            lax.fori_loop(0, _CHUNK // nl, group, 0)
            if c + 1 < nchunk:
                rcopies = issue(c + 1)

        pltpu.sync_copy(outv, out_h.at[pl.ds(base, bpw)])

    return run(users2, movies2, U2, M2, bu, bm, b0v)


# speedup vs baseline: 1.0247x; 1.0247x over previous
import functools

import jax
import jax.numpy as jnp
from jax import lax
from jax.experimental import pallas as pl
from jax.experimental.pallas import tpu as pltpu
from jax.experimental.pallas import tpu_sc as plsc

_BATCH = 16384
_LATENT = 64
_CHUNK = 128


def kernel(users, movies, U, M, bu, bm, b0):
    info = plsc.get_sparse_core_info()
    nc, ns, nl = info.num_cores, info.num_subcores, info.num_lanes
    nw = nc * ns
    bpw = _BATCH // nw
    nchunk = bpw // _CHUNK

    mesh = plsc.VectorSubcoreMesh(core_axis_name="c", subcore_axis_name="s")
    b0v = jnp.broadcast_to(b0.astype(jnp.float32), (nl,))

    @functools.partial(
        pl.kernel,
        mesh=mesh,
        compiler_params=pltpu.CompilerParams(use_tc_tiling_on_sc=False),
        out_type=jax.ShapeDtypeStruct((_BATCH,), jnp.float32),
        scratch_types=[
            pltpu.VMEM((bpw,), jnp.int32),
            pltpu.VMEM((bpw,), jnp.int32),
            pltpu.VMEM((bpw, _LATENT), jnp.float32),
            pltpu.VMEM((bpw, _LATENT), jnp.float32),
            pltpu.VMEM((bpw,), jnp.float32),
            pltpu.VMEM((bpw,), jnp.float32),
            pltpu.VMEM((nl,), jnp.float32),
            pltpu.VMEM((bpw,), jnp.float32),
            pltpu.SemaphoreType.DMA,
        ],
    )
    def run(users_h, movies_h, U_h, M_h, bu_h, bm_h, b0_h, out_h,
            uidx, midx, urows, mrows, ubias, mbias, b0s, outv, sem):
        wid = lax.axis_index("s") * nc + lax.axis_index("c")
        base = wid * bpw

        pltpu.sync_copy(users_h.at[pl.ds(base, bpw)], uidx)
        pltpu.sync_copy(movies_h.at[pl.ds(base, bpw)], midx)
        pltpu.sync_copy(b0_h, b0s)

        copies = []
        for i in range(nchunk):
            sl = pl.ds(i * _CHUNK, _CHUNK)
            copies.append(pltpu.async_copy(U_h.at[uidx.at[sl]], urows.at[sl], sem))
            copies.append(pltpu.async_copy(M_h.at[midx.at[sl]], mrows.at[sl], sem))
            copies.append(pltpu.async_copy(bu_h.at[uidx.at[sl]], ubias.at[sl], sem))
            copies.append(pltpu.async_copy(bm_h.at[midx.at[sl]], mbias.at[sl], sem))
        for c in copies:
            c.wait()

        b0vec = b0s[...]
        even = jnp.arange(0, 2 * nl, 2, jnp.int32) % nl
        odd = even + 1
        lane_lo = lax.broadcasted_iota(jnp.int32, (nl,), 0) < (nl // 2)

        def shuf(a, idx):
            return a.at[idx].get(mode="promise_in_bounds")

        def hadd(a, b):
            ha = shuf(a, even) + shuf(a, odd)
            hb = shuf(b, even) + shuf(b, odd)
            return jnp.where(lane_lo, ha, hb)

        def group(g, carry):
            gbase = g * nl
            vecs = []
            for jj in range(nl):
                p = gbase + jj
                acc = urows[p, pl.ds(0, nl)] * mrows[p, pl.ds(0, nl)]
                for k in range(1, _LATENT // nl):
                    acc = acc + (urows[p, pl.ds(k * nl, nl)]
                                 * mrows[p, pl.ds(k * nl, nl)])
                vecs.append(acc)
            while len(vecs) > 1:
                vecs = [hadd(vecs[t], vecs[t + 1]) for t in range(0, len(vecs), 2)]
            sl = pl.ds(gbase, nl)
            r = vecs[0] + ubias[sl] + mbias[sl] + b0vec
            outv[sl] = 5.0 / (1.0 + jnp.exp(-r))
            return carry

        lax.fori_loop(0, bpw // nl, group, 0)
        pltpu.sync_copy(outv, out_h.at[pl.ds(base, bpw)])

    return run(users.astype(jnp.int32), movies.astype(jnp.int32),
               U, M, bu, bm, b0v)
